# baseline (speedup 1.0000x reference)
import functools

import jax
import jax.numpy as jnp
from jax import lax
from jax.experimental import pallas as pl
from jax.experimental.pallas import tpu as pltpu
from jax.experimental.pallas import tpu_sc as plsc

N = 10000
E = 320000
D = 128

NC = 2
NS = 16
NW = NC * NS
EPW = E // NW
CH = 125
NCHUNK = EPW // CH
HCHUNK = NCHUNK // 2
NPAD = 10240
ROWS_PER_TILE = NPAD // NS


def _agg_body(hs_hbm, src3_hbm, dst3_hbm, zeros_hbm, out_hbm,
              sidx, didx, data0, data1, sem0, sem1, acc_sh):
    cid = lax.axis_index("c")
    sid = lax.axis_index("s")
    wid = sid * NC + cid
    r0 = sid * ROWS_PER_TILE
    pltpu.sync_copy(zeros_hbm, acc_sh.at[pl.ds(r0, ROWS_PER_TILE), :])
    plsc.subcore_barrier()

    for h in range(2):
        pltpu.sync_copy(src3_hbm.at[wid, pl.ds(h * HCHUNK, HCHUNK)], sidx)
        pltpu.sync_copy(dst3_hbm.at[wid, pl.ds(h * HCHUNK, HCHUNK)], didx)
        pltpu.async_copy(hs_hbm.at[sidx.at[0]], data0, sem0)

        def body(i, carry):
            q0 = 2 * i
            pltpu.async_copy(hs_hbm.at[sidx.at[q0 + 1]], data1, sem1)
            pltpu.make_async_copy(hs_hbm.at[sidx.at[q0]], data0, sem0).wait()
            pltpu.sync_copy(data0, acc_sh.at[didx.at[q0]], add=True)

            @pl.when(q0 + 2 < HCHUNK)
            def _():
                pltpu.async_copy(hs_hbm.at[sidx.at[q0 + 2]], data0, sem0)

            pltpu.make_async_copy(hs_hbm.at[sidx.at[q0 + 1]], data1, sem1).wait()
            pltpu.sync_copy(data1, acc_sh.at[didx.at[q0 + 1]], add=True)
            return carry

        lax.fori_loop(0, HCHUNK // 2, body, 0)
    plsc.subcore_barrier()
    pltpu.sync_copy(acc_sh.at[pl.ds(r0, ROWS_PER_TILE), :],
                    out_hbm.at[cid, pl.ds(r0, ROWS_PER_TILE), :])


def _deg_body(dst3_hbm, ones_hbm, zeros_hbm, out_hbm, didx, ones_v, acc_sh):
    cid = lax.axis_index("c")
    sid = lax.axis_index("s")
    wid = sid * NC + cid
    pltpu.sync_copy(dst3_hbm.at[wid], didx)
    pltpu.sync_copy(ones_hbm, ones_v)
    r0 = sid * ROWS_PER_TILE
    pltpu.sync_copy(zeros_hbm, acc_sh.at[pl.ds(r0, ROWS_PER_TILE), :])
    plsc.subcore_barrier()

    def body(q, carry):
        pltpu.sync_copy(ones_v, acc_sh.at[didx.at[q]], add=True)
        return carry

    lax.fori_loop(0, NCHUNK, body, 0)
    plsc.subcore_barrier()
    pltpu.sync_copy(acc_sh.at[pl.ds(r0, ROWS_PER_TILE), :],
                    out_hbm.at[cid, pl.ds(r0, ROWS_PER_TILE), :])


@functools.cache
def _sc_kernels():
    mesh = plsc.VectorSubcoreMesh(core_axis_name="c", subcore_axis_name="s",
                                  num_cores=NC, num_subcores=NS)
    agg_kernel = pl.kernel(
        _agg_body,
        out_type=jax.ShapeDtypeStruct((NC, NPAD, D), jnp.float32),
        mesh=mesh,
        scratch_types=[
            pltpu.VMEM((HCHUNK, CH), jnp.int32),
            pltpu.VMEM((HCHUNK, CH), jnp.int32),
            pltpu.VMEM((CH, D), jnp.float32),
            pltpu.VMEM((CH, D), jnp.float32),
            pltpu.SemaphoreType.DMA,
            pltpu.SemaphoreType.DMA,
            pltpu.VMEM_SHARED((NPAD, D), jnp.float32),
        ],
    )
    deg_kernel = pl.kernel(
        _deg_body,
        out_type=jax.ShapeDtypeStruct((NC, NPAD, D), jnp.float32),
        mesh=mesh,
        scratch_types=[
            pltpu.VMEM((NCHUNK, CH), jnp.int32),
            pltpu.VMEM((CH, D), jnp.float32),
            pltpu.VMEM_SHARED((NPAD, D), jnp.float32),
        ],
    )
    return agg_kernel, deg_kernel


BM = 1000
NBLK = N // BM


def _dinv_block(deg_ref):
    d = deg_ref[0, :, 0:1] + deg_ref[1, :, 0:1]
    return lax.rsqrt(jnp.maximum(d, 1.0))


def _ka_body(feat_ref, w0_ref, w1_ref, u0_ref, v0_ref):
    u0 = jnp.dot(feat_ref[...], w0_ref[...], preferred_element_type=jnp.float32, precision=lax.Precision.HIGHEST)
    u0_ref[...] = u0
    v0_ref[...] = jnp.dot(u0, w1_ref[...], preferred_element_type=jnp.float32, precision=lax.Precision.HIGHEST)


def _kb_body(u0_ref, deg_ref, hs0_ref):
    hs0_ref[...] = u0_ref[...] * _dinv_block(deg_ref)


def _kd_body(v0_ref, p_ref, deg_ref, w1_ref, wp1_ref, bp1_ref,
             hs1_ref, m0_ref):
    dinv = _dinv_block(deg_ref)
    agg = p_ref[0] + p_ref[1]
    h1 = v0_ref[...] + dinv * jnp.dot(agg, w1_ref[...],
                                      preferred_element_type=jnp.float32, precision=lax.Precision.HIGHEST)
    hs1_ref[...] = h1 * dinv
    m0_ref[...] = (jnp.dot(h1, wp1_ref[...], preferred_element_type=jnp.float32, precision=lax.Precision.HIGHEST)
                   + bp1_ref[...])


def _ke_body(m0_ref, p_ref, deg_ref, wp1_ref, m_ref, colsum_ref, sqsum_ref):
    dinv = _dinv_block(deg_ref)
    agg = p_ref[0] + p_ref[1]
    m = m0_ref[...] + dinv * jnp.dot(agg, wp1_ref[...],
                                     preferred_element_type=jnp.float32, precision=lax.Precision.HIGHEST)
    m_ref[...] = m

    @pl.when(pl.program_id(0) == 0)
    def _():
        colsum_ref[...] = jnp.zeros_like(colsum_ref)
        sqsum_ref[...] = jnp.zeros_like(sqsum_ref)

    colsum_ref[...] += jnp.sum(m, axis=0, keepdims=True)
    sqsum_ref[...] += jnp.sum(m * m, axis=0, keepdims=True)


def _kf_body(m_ref, colsum_ref, sqsum_ref, gamma_ref, beta_ref,
             w_ref, b_ref, out_ref):
    mu = colsum_ref[...] * (1.0 / N)
    var = sqsum_ref[...] * (1.0 / N) - mu * mu
    xh = (m_ref[...] - mu) * lax.rsqrt(var + 1e-5) * gamma_ref[...] + beta_ref[...]
    r = jnp.maximum(xh, 0.0)
    out_ref[...] = (jnp.dot(r, w_ref[...], preferred_element_type=jnp.float32, precision=lax.Precision.HIGHEST)
                    + b_ref[...])


def _row_spec(width):
    return pl.BlockSpec((BM, width), lambda i: (i, 0))


def _full_spec(shape):
    return pl.BlockSpec(shape, lambda i: tuple(0 for _ in shape))


def _part_spec(width):
    return pl.BlockSpec((NC, BM, width), lambda i: (0, i, 0))


def kernel(feat, edge_index, W0, W1, Wp1, bp1, gamma, beta, Wp2, bp2):
    src3 = edge_index[0].reshape(NW, NCHUNK, CH)
    dst3 = edge_index[1].reshape(NW, NCHUNK, CH)
    zeros_blk = jnp.zeros((ROWS_PER_TILE, D), jnp.float32)
    ones_blk = jnp.ones((CH, D), jnp.float32)

    agg_kernel, deg_kernel = _sc_kernels()

    u0, v0 = pl.pallas_call(
        _ka_body,
        grid=(NBLK,),
        in_specs=[_row_spec(D), _full_spec((D, D)), _full_spec((D, D))],
        out_specs=[_row_spec(D), _row_spec(D)],
        out_shape=[jax.ShapeDtypeStruct((N, D), jnp.float32)] * 2,
    )(feat, W0, W1)

    deg16 = deg_kernel(dst3, ones_blk, zeros_blk)

    hs0 = pl.pallas_call(
        _kb_body,
        grid=(NBLK,),
        in_specs=[_row_spec(D), _part_spec(D)],
        out_specs=_row_spec(D),
        out_shape=jax.ShapeDtypeStruct((N, D), jnp.float32),
    )(u0, deg16)

    p1 = agg_kernel(hs0, src3, dst3, zeros_blk)

    MH = Wp1.shape[1]
    hs1, m0 = pl.pallas_call(
        _kd_body,
        grid=(NBLK,),
        in_specs=[_row_spec(D), _part_spec(D), _part_spec(D),
                  _full_spec((D, D)), _full_spec((D, MH)),
                  _full_spec((1, MH))],
        out_specs=[_row_spec(D), _row_spec(MH)],
        out_shape=[jax.ShapeDtypeStruct((N, D), jnp.float32),
                   jax.ShapeDtypeStruct((N, MH), jnp.float32)],
    )(v0, p1, deg16, W1, Wp1, bp1.reshape(1, MH))

    p2 = agg_kernel(hs1, src3, dst3, zeros_blk)

    m, colsum, sqsum = pl.pallas_call(
        _ke_body,
        grid=(NBLK,),
        in_specs=[_row_spec(MH), _part_spec(D), _part_spec(D),
                  _full_spec((D, MH))],
        out_specs=[_row_spec(MH), _full_spec((1, MH)), _full_spec((1, MH))],
        out_shape=[jax.ShapeDtypeStruct((N, MH), jnp.float32),
                   jax.ShapeDtypeStruct((1, MH), jnp.float32),
                   jax.ShapeDtypeStruct((1, MH), jnp.float32)],
    )(m0, p2, deg16, Wp1)

    MO = Wp2.shape[1]
    out = pl.pallas_call(
        _kf_body,
        grid=(NBLK,),
        in_specs=[_row_spec(MH), _full_spec((1, MH)), _full_spec((1, MH)),
                  _full_spec((1, MH)), _full_spec((1, MH)),
                  _full_spec((MH, MO)), _full_spec((1, MO))],
        out_specs=_row_spec(MO),
        out_shape=jax.ShapeDtypeStruct((N, MO), jnp.float32),
    )(m, colsum, sqsum, gamma.reshape(1, MH), beta.reshape(1, MH),
      Wp2, bp2.reshape(1, MO))

    return out


# speedup vs baseline: 1.0794x; 1.0794x over previous
import functools

import jax
import jax.numpy as jnp
from jax import lax
from jax.experimental import pallas as pl
from jax.experimental.pallas import tpu as pltpu
from jax.experimental.pallas import tpu_sc as plsc

N = 10000
E = 320000
D = 128

NC = 2
NS = 16
NW = NC * NS
EPW = E // NW
CH = 125
NCHUNK = EPW // CH
HCHUNK = NCHUNK // 2
NPAD = 10240
ROWS_PER_TILE = NPAD // NS


def _agg_body(hs_hbm, src3_hbm, dst3_hbm, zeros_hbm, out_hbm,
              sidx, didx, data0, data1, sem0, sem1, acc_sh):
    cid = lax.axis_index("c")
    sid = lax.axis_index("s")
    wid = sid * NC + cid
    r0 = sid * ROWS_PER_TILE
    pltpu.sync_copy(zeros_hbm, acc_sh.at[pl.ds(r0, ROWS_PER_TILE), :])
    plsc.subcore_barrier()

    for h in range(2):
        pltpu.sync_copy(src3_hbm.at[wid, pl.ds(h * HCHUNK, HCHUNK)], sidx)
        pltpu.sync_copy(dst3_hbm.at[wid, pl.ds(h * HCHUNK, HCHUNK)], didx)
        pltpu.async_copy(hs_hbm.at[sidx.at[0]], data0, sem0)

        def body(i, carry):
            q0 = 2 * i
            pltpu.async_copy(hs_hbm.at[sidx.at[q0 + 1]], data1, sem1)
            pltpu.make_async_copy(hs_hbm.at[sidx.at[q0]], data0, sem0).wait()
            pltpu.sync_copy(data0, acc_sh.at[didx.at[q0]], add=True)

            @pl.when(q0 + 2 < HCHUNK)
            def _():
                pltpu.async_copy(hs_hbm.at[sidx.at[q0 + 2]], data0, sem0)

            pltpu.make_async_copy(hs_hbm.at[sidx.at[q0 + 1]], data1, sem1).wait()
            pltpu.sync_copy(data1, acc_sh.at[didx.at[q0 + 1]], add=True)
            return carry

        lax.fori_loop(0, HCHUNK // 2, body, 0)
    plsc.subcore_barrier()
    pltpu.sync_copy(acc_sh.at[pl.ds(r0, ROWS_PER_TILE), :],
                    out_hbm.at[cid, pl.ds(r0, ROWS_PER_TILE), :])


def _deg_body(dst3_hbm, ones_hbm, zeros_hbm, out_hbm, didx, ones_v, acc_sh):
    cid = lax.axis_index("c")
    sid = lax.axis_index("s")
    wid = sid * NC + cid
    pltpu.sync_copy(dst3_hbm.at[wid], didx)
    pltpu.sync_copy(ones_hbm, ones_v)
    r0 = sid * ROWS_PER_TILE
    pltpu.sync_copy(zeros_hbm, acc_sh.at[pl.ds(r0, ROWS_PER_TILE), :])
    plsc.subcore_barrier()

    def body(q, carry):
        pltpu.sync_copy(ones_v, acc_sh.at[didx.at[q]], add=True)
        return carry

    lax.fori_loop(0, NCHUNK, body, 0)
    plsc.subcore_barrier()
    pltpu.sync_copy(acc_sh.at[pl.ds(r0, ROWS_PER_TILE), :],
                    out_hbm.at[cid, pl.ds(r0, ROWS_PER_TILE), :])


@functools.cache
def _sc_kernels():
    mesh = plsc.VectorSubcoreMesh(core_axis_name="c", subcore_axis_name="s",
                                  num_cores=NC, num_subcores=NS)
    agg_kernel = pl.kernel(
        _agg_body,
        out_type=jax.ShapeDtypeStruct((NC, NPAD, D), jnp.float32),
        mesh=mesh,
        scratch_types=[
            pltpu.VMEM((HCHUNK, CH), jnp.int32),
            pltpu.VMEM((HCHUNK, CH), jnp.int32),
            pltpu.VMEM((CH, D), jnp.float32),
            pltpu.VMEM((CH, D), jnp.float32),
            pltpu.SemaphoreType.DMA,
            pltpu.SemaphoreType.DMA,
            pltpu.VMEM_SHARED((NPAD, D), jnp.float32),
        ],
    )
    deg_kernel = pl.kernel(
        _deg_body,
        out_type=jax.ShapeDtypeStruct((NC, NPAD, D), jnp.float32),
        mesh=mesh,
        scratch_types=[
            pltpu.VMEM((NCHUNK, CH), jnp.int32),
            pltpu.VMEM((CH, D), jnp.float32),
            pltpu.VMEM_SHARED((NPAD, D), jnp.float32),
        ],
    )
    return agg_kernel, deg_kernel


BM = 1000
NBLK = N // BM


def _deg_block(deg_ref):
    return jnp.maximum(deg_ref[0, :, 0:1] + deg_ref[1, :, 0:1], 1.0)


def _k1_body(feat_ref, w0_ref, deg_ref, hs0_ref):
    h0 = jnp.dot(feat_ref[...], w0_ref[...], preferred_element_type=jnp.float32)
    hs0_ref[...] = h0 * lax.rsqrt(_deg_block(deg_ref))


def _k2_body(hs0_ref, p_ref, deg_ref, w_ref, hs1_ref):
    d = _deg_block(deg_ref)
    dinv = lax.rsqrt(d)
    h0 = hs0_ref[...] * jnp.sqrt(d)
    c = h0 + dinv * (p_ref[0] + p_ref[1])
    h1 = jnp.dot(c, w_ref[...], preferred_element_type=jnp.float32)
    hs1_ref[...] = h1 * dinv


def _k3_body(hs1_ref, p_ref, deg_ref, w_ref, b_ref,
             m_ref, colsum_ref, sqsum_ref):
    d = _deg_block(deg_ref)
    dinv = lax.rsqrt(d)
    h1 = hs1_ref[...] * jnp.sqrt(d)
    c = h1 + dinv * (p_ref[0] + p_ref[1])
    m = jnp.dot(c, w_ref[...], preferred_element_type=jnp.float32) + b_ref[...]
    m_ref[...] = m

    @pl.when(pl.program_id(0) == 0)
    def _():
        colsum_ref[...] = jnp.zeros_like(colsum_ref)
        sqsum_ref[...] = jnp.zeros_like(sqsum_ref)

    colsum_ref[...] += jnp.sum(m, axis=0, keepdims=True)
    sqsum_ref[...] += jnp.sum(m * m, axis=0, keepdims=True)


def _kf_body(m_ref, colsum_ref, sqsum_ref, gamma_ref, beta_ref,
             w_ref, b_ref, out_ref):
    mu = colsum_ref[...] * (1.0 / N)
    var = sqsum_ref[...] * (1.0 / N) - mu * mu
    xh = (m_ref[...] - mu) * lax.rsqrt(var + 1e-5) * gamma_ref[...] + beta_ref[...]
    r = jnp.maximum(xh, 0.0)
    out_ref[...] = (jnp.dot(r, w_ref[...], preferred_element_type=jnp.float32)
                    + b_ref[...])


def _row_spec(width):
    return pl.BlockSpec((BM, width), lambda i: (i, 0))


def _full_spec(shape):
    return pl.BlockSpec(shape, lambda i: tuple(0 for _ in shape))


def _part_spec(width):
    return pl.BlockSpec((NC, BM, width), lambda i: (0, i, 0))


def kernel(feat, edge_index, W0, W1, Wp1, bp1, gamma, beta, Wp2, bp2):
    src3 = edge_index[0].reshape(NW, NCHUNK, CH)
    dst3 = edge_index[1].reshape(NW, NCHUNK, CH)
    zeros_blk = jnp.zeros((ROWS_PER_TILE, D), jnp.float32)
    ones_blk = jnp.ones((CH, D), jnp.float32)

    agg_kernel, deg_kernel = _sc_kernels()

    deg16 = deg_kernel(dst3, ones_blk, zeros_blk)

    hs0 = pl.pallas_call(
        _k1_body,
        grid=(NBLK,),
        in_specs=[_row_spec(D), _full_spec((D, D)), _part_spec(D)],
        out_specs=_row_spec(D),
        out_shape=jax.ShapeDtypeStruct((N, D), jnp.float32),
    )(feat, W0, deg16)

    p1 = agg_kernel(hs0, src3, dst3, zeros_blk)

    hs1 = pl.pallas_call(
        _k2_body,
        grid=(NBLK,),
        in_specs=[_row_spec(D), _part_spec(D), _part_spec(D),
                  _full_spec((D, D))],
        out_specs=_row_spec(D),
        out_shape=jax.ShapeDtypeStruct((N, D), jnp.float32),
    )(hs0, p1, deg16, W1)

    p2 = agg_kernel(hs1, src3, dst3, zeros_blk)

    MH = Wp1.shape[1]
    m, colsum, sqsum = pl.pallas_call(
        _k3_body,
        grid=(NBLK,),
        in_specs=[_row_spec(D), _part_spec(D), _part_spec(D),
                  _full_spec((D, MH)), _full_spec((1, MH))],
        out_specs=[_row_spec(MH), _full_spec((1, MH)), _full_spec((1, MH))],
        out_shape=[jax.ShapeDtypeStruct((N, MH), jnp.float32),
                   jax.ShapeDtypeStruct((1, MH), jnp.float32),
                   jax.ShapeDtypeStruct((1, MH), jnp.float32)],
    )(hs1, p2, deg16, Wp1, bp1.reshape(1, MH))

    MO = Wp2.shape[1]
    out = pl.pallas_call(
        _kf_body,
        grid=(NBLK,),
        in_specs=[_row_spec(MH), _full_spec((1, MH)), _full_spec((1, MH)),
                  _full_spec((1, MH)), _full_spec((1, MH)),
                  _full_spec((MH, MO)), _full_spec((1, MO))],
        out_specs=_row_spec(MO),
        out_shape=jax.ShapeDtypeStruct((N, MO), jnp.float32),
    )(m, colsum, sqsum, gamma.reshape(1, MH), beta.reshape(1, MH),
      Wp2, bp2.reshape(1, MO))

    return out


# speedup vs baseline: 1.1031x; 1.0220x over previous
import functools

import jax
import jax.numpy as jnp
from jax import lax
from jax.experimental import pallas as pl
from jax.experimental.pallas import tpu as pltpu
from jax.experimental.pallas import tpu_sc as plsc

N = 10000
E = 320000
D = 128

NC = 2
NS = 16
NW = NC * NS
EPW = E // NW
CH = 125
NCHUNK = EPW // CH
HCHUNK = NCHUNK // 2
NPAD = 10240
ROWS_PER_TILE = NPAD // NS


def _agg_body(hs_hbm, src3_hbm, dst3_hbm, zeros_hbm, out_hbm,
              sidx, didx, data0, data1, sem0, sem1, acc_sh):
    cid = lax.axis_index("c")
    sid = lax.axis_index("s")
    wid = sid * NC + cid
    r0 = sid * ROWS_PER_TILE
    pltpu.sync_copy(zeros_hbm, acc_sh.at[pl.ds(r0, ROWS_PER_TILE), :])
    plsc.subcore_barrier()

    for h in range(2):
        pltpu.sync_copy(src3_hbm.at[wid, pl.ds(h * HCHUNK, HCHUNK)], sidx)
        pltpu.sync_copy(dst3_hbm.at[wid, pl.ds(h * HCHUNK, HCHUNK)], didx)
        pltpu.async_copy(hs_hbm.at[sidx.at[0]], data0, sem0)

        def body(i, carry):
            q0 = 2 * i
            pltpu.async_copy(hs_hbm.at[sidx.at[q0 + 1]], data1, sem1)
            pltpu.make_async_copy(hs_hbm.at[sidx.at[q0]], data0, sem0).wait()
            pltpu.sync_copy(data0, acc_sh.at[didx.at[q0]], add=True)

            @pl.when(q0 + 2 < HCHUNK)
            def _():
                pltpu.async_copy(hs_hbm.at[sidx.at[q0 + 2]], data0, sem0)

            pltpu.make_async_copy(hs_hbm.at[sidx.at[q0 + 1]], data1, sem1).wait()
            pltpu.sync_copy(data1, acc_sh.at[didx.at[q0 + 1]], add=True)
            return carry

        lax.fori_loop(0, HCHUNK // 2, body, 0)
    plsc.subcore_barrier()
    pltpu.sync_copy(acc_sh.at[pl.ds(r0, ROWS_PER_TILE), :],
                    out_hbm.at[cid, pl.ds(r0, ROWS_PER_TILE), :])


def _deg_body(dst3_hbm, ones_hbm, zeros_hbm, out_hbm, didx, ones_v, acc_sh):
    cid = lax.axis_index("c")
    sid = lax.axis_index("s")
    wid = sid * NC + cid
    pltpu.sync_copy(dst3_hbm.at[wid], didx)
    pltpu.sync_copy(ones_hbm, ones_v)
    r0 = sid * ROWS_PER_TILE
    pltpu.sync_copy(zeros_hbm, acc_sh.at[pl.ds(r0, ROWS_PER_TILE), :])
    plsc.subcore_barrier()

    def body(q, carry):
        pltpu.sync_copy(ones_v, acc_sh.at[didx.at[q]], add=True)
        return carry

    lax.fori_loop(0, NCHUNK, body, 0)
    plsc.subcore_barrier()
    pltpu.sync_copy(acc_sh.at[pl.ds(r0, ROWS_PER_TILE), :],
                    out_hbm.at[cid, pl.ds(r0, ROWS_PER_TILE), :])


@functools.cache
def _sc_kernels():
    mesh = plsc.VectorSubcoreMesh(core_axis_name="c", subcore_axis_name="s",
                                  num_cores=NC, num_subcores=NS)
    agg_kernel = pl.kernel(
        _agg_body,
        out_type=jax.ShapeDtypeStruct((NC, NPAD, D), jnp.float32),
        mesh=mesh,
        scratch_types=[
            pltpu.VMEM((HCHUNK, CH), jnp.int32),
            pltpu.VMEM((HCHUNK, CH), jnp.int32),
            pltpu.VMEM((CH, D), jnp.float32),
            pltpu.VMEM((CH, D), jnp.float32),
            pltpu.SemaphoreType.DMA,
            pltpu.SemaphoreType.DMA,
            pltpu.VMEM_SHARED((NPAD, D), jnp.float32),
        ],
    )
    deg_kernel = pl.kernel(
        _deg_body,
        out_type=jax.ShapeDtypeStruct((NC, NPAD, D), jnp.float32),
        mesh=mesh,
        scratch_types=[
            pltpu.VMEM((NCHUNK, CH), jnp.int32),
            pltpu.VMEM((CH, D), jnp.float32),
            pltpu.VMEM_SHARED((NPAD, D), jnp.float32),
        ],
    )
    return agg_kernel, deg_kernel


BM = 2000
NBLK = N // BM


def _deg_block(deg_ref):
    return jnp.maximum(deg_ref[0, :, 0:1] + deg_ref[1, :, 0:1], 1.0)


def _k1_body(feat_ref, w0_ref, deg_ref, hs0_ref):
    h0 = jnp.dot(feat_ref[...], w0_ref[...], preferred_element_type=jnp.float32)
    hs0_ref[...] = h0 * lax.rsqrt(_deg_block(deg_ref))


def _k2_body(hs0_ref, p_ref, deg_ref, w_ref, hs1_ref):
    d = _deg_block(deg_ref)
    dinv = lax.rsqrt(d)
    h0 = hs0_ref[...] * jnp.sqrt(d)
    c = h0 + dinv * (p_ref[0] + p_ref[1])
    h1 = jnp.dot(c, w_ref[...], preferred_element_type=jnp.float32)
    hs1_ref[...] = h1 * dinv


def _k3_body(hs1_ref, p_ref, deg_ref, w_ref, b_ref,
             m_ref, colsum_ref, sqsum_ref):
    d = _deg_block(deg_ref)
    dinv = lax.rsqrt(d)
    h1 = hs1_ref[...] * jnp.sqrt(d)
    c = h1 + dinv * (p_ref[0] + p_ref[1])
    m = jnp.dot(c, w_ref[...], preferred_element_type=jnp.float32) + b_ref[...]
    m_ref[...] = m

    @pl.when(pl.program_id(0) == 0)
    def _():
        colsum_ref[...] = jnp.zeros_like(colsum_ref)
        sqsum_ref[...] = jnp.zeros_like(sqsum_ref)

    colsum_ref[...] += jnp.sum(m, axis=0, keepdims=True)
    sqsum_ref[...] += jnp.sum(m * m, axis=0, keepdims=True)


def _kf_body(m_ref, colsum_ref, sqsum_ref, gamma_ref, beta_ref,
             w_ref, b_ref, out_ref):
    mu = colsum_ref[...] * (1.0 / N)
    var = sqsum_ref[...] * (1.0 / N) - mu * mu
    xh = (m_ref[...] - mu) * lax.rsqrt(var + 1e-5) * gamma_ref[...] + beta_ref[...]
    r = jnp.maximum(xh, 0.0)
    out_ref[...] = (jnp.dot(r, w_ref[...], preferred_element_type=jnp.float32)
                    + b_ref[...])


def _row_spec(width):
    return pl.BlockSpec((BM, width), lambda i: (i, 0))


def _full_spec(shape):
    return pl.BlockSpec(shape, lambda i: tuple(0 for _ in shape))


def _part_spec(width):
    return pl.BlockSpec((NC, BM, width), lambda i: (0, i, 0))


def kernel(feat, edge_index, W0, W1, Wp1, bp1, gamma, beta, Wp2, bp2):
    src3 = edge_index[0].reshape(NW, NCHUNK, CH)
    dst3 = edge_index[1].reshape(NW, NCHUNK, CH)
    zeros_blk = jnp.zeros((ROWS_PER_TILE, D), jnp.float32)
    ones_blk = jnp.ones((CH, D), jnp.float32)

    agg_kernel, deg_kernel = _sc_kernels()

    deg16 = deg_kernel(dst3, ones_blk, zeros_blk)

    hs0 = pl.pallas_call(
        _k1_body,
        grid=(NBLK,),
        in_specs=[_row_spec(D), _full_spec((D, D)), _part_spec(D)],
        out_specs=_row_spec(D),
        out_shape=jax.ShapeDtypeStruct((N, D), jnp.float32),
    )(feat, W0, deg16)

    p1 = agg_kernel(hs0, src3, dst3, zeros_blk)

    hs1 = pl.pallas_call(
        _k2_body,
        grid=(NBLK,),
        in_specs=[_row_spec(D), _part_spec(D), _part_spec(D),
                  _full_spec((D, D))],
        out_specs=_row_spec(D),
        out_shape=jax.ShapeDtypeStruct((N, D), jnp.float32),
    )(hs0, p1, deg16, W1)

    p2 = agg_kernel(hs1, src3, dst3, zeros_blk)

    MH = Wp1.shape[1]
    m, colsum, sqsum = pl.pallas_call(
        _k3_body,
        grid=(NBLK,),
        in_specs=[_row_spec(D), _part_spec(D), _part_spec(D),
                  _full_spec((D, MH)), _full_spec((1, MH))],
        out_specs=[_row_spec(MH), _full_spec((1, MH)), _full_spec((1, MH))],
        out_shape=[jax.ShapeDtypeStruct((N, MH), jnp.float32),
                   jax.ShapeDtypeStruct((1, MH), jnp.float32),
                   jax.ShapeDtypeStruct((1, MH), jnp.float32)],
    )(hs1, p2, deg16, Wp1, bp1.reshape(1, MH))

    MO = Wp2.shape[1]
    out = pl.pallas_call(
        _kf_body,
        grid=(NBLK,),
        in_specs=[_row_spec(MH), _full_spec((1, MH)), _full_spec((1, MH)),
                  _full_spec((1, MH)), _full_spec((1, MH)),
                  _full_spec((MH, MO)), _full_spec((1, MO))],
        out_specs=_row_spec(MO),
        out_shape=jax.ShapeDtypeStruct((N, MO), jnp.float32),
    )(m, colsum, sqsum, gamma.reshape(1, MH), beta.reshape(1, MH),
      Wp2, bp2.reshape(1, MO))

    return out


# speedup vs baseline: 1.1151x; 1.0108x over previous
import functools

import jax
import jax.numpy as jnp
from jax import lax
from jax.experimental import pallas as pl
from jax.experimental.pallas import tpu as pltpu
from jax.experimental.pallas import tpu_sc as plsc

N = 10000
E = 320000
D = 128

NC = 2
NS = 16
NW = NC * NS
EPW = E // NW
CH = 125
NCHUNK = EPW // CH
HCHUNK = NCHUNK // 2
NPAD = 10240
ROWS_PER_TILE = NPAD // NS


def _agg_body(hs_hbm, src3_hbm, dst3_hbm, zeros_hbm, out_hbm,
              sidx, didx, data0, data1, sem0, sem1, acc_sh):
    cid = lax.axis_index("c")
    sid = lax.axis_index("s")
    wid = sid * NC + cid
    r0 = sid * ROWS_PER_TILE
    pltpu.sync_copy(zeros_hbm, acc_sh.at[pl.ds(r0, ROWS_PER_TILE), :])
    plsc.subcore_barrier()

    for h in range(2):
        pltpu.sync_copy(src3_hbm.at[wid, pl.ds(h * HCHUNK, HCHUNK)], sidx)
        pltpu.sync_copy(dst3_hbm.at[wid, pl.ds(h * HCHUNK, HCHUNK)], didx)
        pltpu.async_copy(hs_hbm.at[sidx.at[0]], data0, sem0)

        def body(i, carry):
            q0 = 2 * i
            pltpu.async_copy(hs_hbm.at[sidx.at[q0 + 1]], data1, sem1)
            pltpu.make_async_copy(hs_hbm.at[sidx.at[q0]], data0, sem0).wait()
            pltpu.sync_copy(data0, acc_sh.at[didx.at[q0]], add=True)

            @pl.when(q0 + 2 < HCHUNK)
            def _():
                pltpu.async_copy(hs_hbm.at[sidx.at[q0 + 2]], data0, sem0)

            pltpu.make_async_copy(hs_hbm.at[sidx.at[q0 + 1]], data1, sem1).wait()
            pltpu.sync_copy(data1, acc_sh.at[didx.at[q0 + 1]], add=True)
            return carry

        lax.fori_loop(0, HCHUNK // 2, body, 0)
    plsc.subcore_barrier()
    pltpu.sync_copy(acc_sh.at[pl.ds(r0, ROWS_PER_TILE), :],
                    out_hbm.at[cid, pl.ds(r0, ROWS_PER_TILE), :])


def _deg_body(dst3_hbm, ones_hbm, zeros_hbm, out_hbm, didx, ones_v, acc_sh):
    cid = lax.axis_index("c")
    sid = lax.axis_index("s")
    wid = sid * NC + cid
    pltpu.sync_copy(dst3_hbm.at[wid], didx)
    pltpu.sync_copy(ones_hbm, ones_v)
    r0 = sid * ROWS_PER_TILE
    pltpu.sync_copy(zeros_hbm, acc_sh.at[pl.ds(r0, ROWS_PER_TILE), :])
    plsc.subcore_barrier()

    def body(q, carry):
        pltpu.sync_copy(ones_v, acc_sh.at[didx.at[q]], add=True)
        return carry

    lax.fori_loop(0, NCHUNK, body, 0)
    plsc.subcore_barrier()
    pltpu.sync_copy(acc_sh.at[pl.ds(r0, ROWS_PER_TILE), :],
                    out_hbm.at[cid, pl.ds(r0, ROWS_PER_TILE), :])


@functools.cache
def _sc_kernels():
    mesh = plsc.VectorSubcoreMesh(core_axis_name="c", subcore_axis_name="s",
                                  num_cores=NC, num_subcores=NS)
    agg_kernel = pl.kernel(
        _agg_body,
        out_type=jax.ShapeDtypeStruct((NC, NPAD, D), jnp.float32),
        mesh=mesh,
        scratch_types=[
            pltpu.VMEM((HCHUNK, CH), jnp.int32),
            pltpu.VMEM((HCHUNK, CH), jnp.int32),
            pltpu.VMEM((CH, D), jnp.float32),
            pltpu.VMEM((CH, D), jnp.float32),
            pltpu.SemaphoreType.DMA,
            pltpu.SemaphoreType.DMA,
            pltpu.VMEM_SHARED((NPAD, D), jnp.float32),
        ],
    )
    deg_kernel = pl.kernel(
        _deg_body,
        out_type=jax.ShapeDtypeStruct((NC, NPAD, D), jnp.float32),
        mesh=mesh,
        scratch_types=[
            pltpu.VMEM((NCHUNK, CH), jnp.int32),
            pltpu.VMEM((CH, D), jnp.float32),
            pltpu.VMEM_SHARED((NPAD, D), jnp.float32),
        ],
    )
    return agg_kernel, deg_kernel


BM = 2000
NBLK = N // BM


def _deg_block(deg_ref):
    return jnp.maximum(deg_ref[0, :, 0:1] + deg_ref[1, :, 0:1], 1.0)


def _k1_body(feat_ref, w0_ref, deg_ref, hs0_ref, dinv_ref):
    dinv = lax.rsqrt(_deg_block(deg_ref))
    h0 = jnp.dot(feat_ref[...], w0_ref[...], preferred_element_type=jnp.float32)
    hs0_ref[...] = h0 * dinv
    dinv_ref[...] = jnp.broadcast_to(dinv, dinv_ref.shape)


def _k2_body(hs0_ref, p_ref, dinv_ref, w_ref, hs1_ref):
    dinv = dinv_ref[...]
    h0 = hs0_ref[...] * (1.0 / dinv)
    c = h0 + dinv * (p_ref[0] + p_ref[1])
    h1 = jnp.dot(c, w_ref[...], preferred_element_type=jnp.float32)
    hs1_ref[...] = h1 * dinv


def _k3_body(hs1_ref, p_ref, dinv_ref, w_ref, b_ref,
             m_ref, colsum_ref, sqsum_ref):
    dinv = dinv_ref[...]
    h1 = hs1_ref[...] * (1.0 / dinv)
    c = h1 + dinv * (p_ref[0] + p_ref[1])
    m = jnp.dot(c, w_ref[...], preferred_element_type=jnp.float32) + b_ref[...]
    m_ref[...] = m

    @pl.when(pl.program_id(0) == 0)
    def _():
        colsum_ref[...] = jnp.zeros_like(colsum_ref)
        sqsum_ref[...] = jnp.zeros_like(sqsum_ref)

    colsum_ref[...] += jnp.sum(m, axis=0, keepdims=True)
    sqsum_ref[...] += jnp.sum(m * m, axis=0, keepdims=True)


def _kf_body(m_ref, colsum_ref, sqsum_ref, gamma_ref, beta_ref,
             w_ref, b_ref, out_ref):
    mu = colsum_ref[...] * (1.0 / N)
    var = sqsum_ref[...] * (1.0 / N) - mu * mu
    xh = (m_ref[...] - mu) * lax.rsqrt(var + 1e-5) * gamma_ref[...] + beta_ref[...]
    r = jnp.maximum(xh, 0.0)
    out_ref[...] = (jnp.dot(r, w_ref[...], preferred_element_type=jnp.float32)
                    + b_ref[...])


def _row_spec(width):
    return pl.BlockSpec((BM, width), lambda i: (i, 0))


def _full_spec(shape):
    return pl.BlockSpec(shape, lambda i: tuple(0 for _ in shape))


def _part_spec(width):
    return pl.BlockSpec((NC, BM, width), lambda i: (0, i, 0))


def kernel(feat, edge_index, W0, W1, Wp1, bp1, gamma, beta, Wp2, bp2):
    src3 = edge_index[0].reshape(NW, NCHUNK, CH)
    dst3 = edge_index[1].reshape(NW, NCHUNK, CH)
    zeros_blk = jnp.zeros((ROWS_PER_TILE, D), jnp.float32)
    ones_blk = jnp.ones((CH, D), jnp.float32)

    agg_kernel, deg_kernel = _sc_kernels()

    deg16 = deg_kernel(dst3, ones_blk, zeros_blk)

    hs0, dinv16 = pl.pallas_call(
        _k1_body,
        grid=(NBLK,),
        in_specs=[_row_spec(D), _full_spec((D, D)), _part_spec(D)],
        out_specs=[_row_spec(D), _row_spec(D)],
        out_shape=[jax.ShapeDtypeStruct((N, D), jnp.float32)] * 2,
    )(feat, W0, deg16)

    p1 = agg_kernel(hs0, src3, dst3, zeros_blk)

    hs1 = pl.pallas_call(
        _k2_body,
        grid=(NBLK,),
        in_specs=[_row_spec(D), _part_spec(D), _row_spec(D),
                  _full_spec((D, D))],
        out_specs=_row_spec(D),
        out_shape=jax.ShapeDtypeStruct((N, D), jnp.float32),
    )(hs0, p1, dinv16, W1)

    p2 = agg_kernel(hs1, src3, dst3, zeros_blk)

    MH = Wp1.shape[1]
    m, colsum, sqsum = pl.pallas_call(
        _k3_body,
        grid=(NBLK,),
        in_specs=[_row_spec(D), _part_spec(D), _row_spec(D),
                  _full_spec((D, MH)), _full_spec((1, MH))],
        out_specs=[_row_spec(MH), _full_spec((1, MH)), _full_spec((1, MH))],
        out_shape=[jax.ShapeDtypeStruct((N, MH), jnp.float32),
                   jax.ShapeDtypeStruct((1, MH), jnp.float32),
                   jax.ShapeDtypeStruct((1, MH), jnp.float32)],
    )(hs1, p2, dinv16, Wp1, bp1.reshape(1, MH))

    MO = Wp2.shape[1]
    out = pl.pallas_call(
        _kf_body,
        grid=(NBLK,),
        in_specs=[_row_spec(MH), _full_spec((1, MH)), _full_spec((1, MH)),
                  _full_spec((1, MH)), _full_spec((1, MH)),
                  _full_spec((MH, MO)), _full_spec((1, MO))],
        out_specs=_row_spec(MO),
        out_shape=jax.ShapeDtypeStruct((N, MO), jnp.float32),
    )(m, colsum, sqsum, gamma.reshape(1, MH), beta.reshape(1, MH),
      Wp2, bp2.reshape(1, MO))

    return out
